# Initial kernel scaffold; baseline (speedup 1.0000x reference)
import jax
import jax.numpy as jnp
from jax.experimental import pallas as pl


def kernel(x, table):
    raise NotImplementedError("write your pallas kernel here")


# speedup vs baseline: 1.1072x; 1.1072x over previous
import functools

import jax
import jax.numpy as jnp
from jax import lax
from jax.experimental import pallas as pl
from jax.experimental.pallas import tpu as pltpu
from jax.experimental.pallas import tpu_sc as plsc

_BATCH = 16384
_HIST = 50
_EMB = 32
_B = _BATCH * _HIST

_NC = 2
_NS = 16
_NW = _NC * _NS
_BPW = _B // _NW
_K = 128
_NCHUNK = _BPW // _K
_G = 8
_NGROUP = _NCHUNK // _G

_mesh = plsc.VectorSubcoreMesh(core_axis_name="c", subcore_axis_name="s")


@functools.partial(
    pl.kernel,
    mesh=_mesh,
    out_type=jax.ShapeDtypeStruct((_B, _EMB), jnp.float32),
    compiler_params=pltpu.CompilerParams(use_tc_tiling_on_sc=False),
    scratch_types=[
        pltpu.VMEM((_NCHUNK, _K), jnp.int32),
        pltpu.VMEM((_G, _K, _EMB), jnp.float32),
        pltpu.SemaphoreType.DMA,
        pltpu.SemaphoreType.DMA,
    ],
)
def _sc_gather(x_hbm, table_hbm, out_hbm, idx_v, rows_v, gsem, osem):
    wid = lax.axis_index("s") * _NC + lax.axis_index("c")
    base = wid * _BPW
    pltpu.sync_copy(x_hbm.at[wid], idx_v)

    def group(g, carry):
        j0 = g * _G
        for b in range(_G):
            pltpu.async_copy(
                table_hbm.at[idx_v.at[j0 + b]], rows_v.at[b], gsem)
        for b in range(_G):
            pltpu.make_async_copy(
                table_hbm.at[idx_v.at[j0 + b]], rows_v.at[b], gsem).wait()
            off = base + (j0 + b) * _K
            pltpu.async_copy(
                rows_v.at[b], out_hbm.at[pl.ds(off, _K)], osem)
        for b in range(_G):
            off = base + (j0 + b) * _K
            pltpu.make_async_copy(
                rows_v.at[b], out_hbm.at[pl.ds(off, _K)], osem).wait()
        return carry

    lax.fori_loop(0, _NGROUP, group, 0)


def _mask_body(x_ref, o_ref):
    o_ref[...] = x_ref[...] > 0


_mask_call = pl.pallas_call(
    _mask_body,
    out_shape=jax.ShapeDtypeStruct((_BATCH, _HIST), jnp.bool_),
    grid=(16,),
    in_specs=[pl.BlockSpec((_BATCH // 16, _HIST), lambda i: (i, 0))],
    out_specs=pl.BlockSpec((_BATCH // 16, _HIST), lambda i: (i, 0)),
)


def kernel(x, table):
    xi = x.astype(jnp.int32)
    x_split = xi.reshape(_NW, _NCHUNK, _K)
    embs = _sc_gather(x_split, table)
    mask = _mask_call(xi)
    return embs.reshape(_BATCH, _HIST, _EMB), mask


# speedup vs baseline: 1.2363x; 1.1166x over previous
import functools

import jax
import jax.numpy as jnp
from jax import lax
from jax.experimental import pallas as pl
from jax.experimental.pallas import tpu as pltpu
from jax.experimental.pallas import tpu_sc as plsc

_BATCH = 16384
_HIST = 50
_EMB = 32
_B = _BATCH * _HIST

_NC = 2
_NS = 16
_NW = _NC * _NS
_BPW = _B // _NW
_K = 128
_NCHUNK = _BPW // _K
_G = 8
_GK = _G * _K
_NGROUP = _NCHUNK // _G
_NB = 3

_mesh = plsc.VectorSubcoreMesh(core_axis_name="c", subcore_axis_name="s")


@functools.partial(
    pl.kernel,
    mesh=_mesh,
    out_type=jax.ShapeDtypeStruct((_NW, _NGROUP, _GK, _EMB), jnp.float32),
    compiler_params=pltpu.CompilerParams(use_tc_tiling_on_sc=False),
    scratch_types=[
        pltpu.VMEM((_NCHUNK, _K), jnp.int32),
        pltpu.VMEM((_NB, _GK, _EMB), jnp.float32),
        pltpu.SemaphoreType.DMA,
        pltpu.SemaphoreType.DMA,
    ],
)
def _sc_gather(x_hbm, table_hbm, out_hbm, idx_v, rows_v, gsem, osem):
    wid = lax.axis_index("s") * _NC + lax.axis_index("c")
    pltpu.sync_copy(x_hbm.at[wid], idx_v)

    def fire_gathers(g, b):
        for k in range(_G):
            pltpu.async_copy(
                table_hbm.at[idx_v.at[g * _G + k]],
                rows_v.at[b, pl.ds(k * _K, _K)], gsem)

    def wait_gathers(g, b):
        for k in range(_G):
            pltpu.make_async_copy(
                table_hbm.at[idx_v.at[g * _G + k]],
                rows_v.at[b, pl.ds(k * _K, _K)], gsem).wait()

    def fire_wb(g, b):
        pltpu.async_copy(rows_v.at[b], out_hbm.at[wid, g], osem)

    def wait_wb(g, b):
        pltpu.make_async_copy(rows_v.at[b], out_hbm.at[wid, g], osem).wait()

    fire_gathers(0, 0)
    fire_gathers(1, 1)
    wait_gathers(0, 0)
    fire_wb(0, 0)
    fire_gathers(2, 2)

    def body(g, carry):
        b = g % _NB
        wait_gathers(g, b)
        wait_wb(g - 1, (g - 1) % _NB)
        fire_wb(g, b)
        fire_gathers(g + 2, (g + 2) % _NB)
        return carry

    lax.fori_loop(1, _NGROUP - 2, body, 0)

    g = _NGROUP - 2
    wait_gathers(g, g % _NB)
    wait_wb(g - 1, (g - 1) % _NB)
    fire_wb(g, g % _NB)
    g = _NGROUP - 1
    wait_gathers(g, g % _NB)
    wait_wb(g - 1, (g - 1) % _NB)
    fire_wb(g, g % _NB)
    wait_wb(g, g % _NB)


def _mask_body(x_ref, o_ref):
    o_ref[...] = x_ref[...] > 0


_mask_call = pl.pallas_call(
    _mask_body,
    out_shape=jax.ShapeDtypeStruct((_BATCH, _HIST), jnp.bool_),
    grid=(16,),
    in_specs=[pl.BlockSpec((_BATCH // 16, _HIST), lambda i: (i, 0))],
    out_specs=pl.BlockSpec((_BATCH // 16, _HIST), lambda i: (i, 0)),
)


def kernel(x, table):
    xi = x.astype(jnp.int32)
    x_split = xi.reshape(_NW, _NCHUNK, _K)
    embs = _sc_gather(x_split, table)
    mask = _mask_call(xi)
    return embs.reshape(_BATCH, _HIST, _EMB), mask


# speedup vs baseline: 1.4981x; 1.2118x over previous
import functools

import jax
import jax.numpy as jnp
from jax import lax
from jax.experimental import pallas as pl
from jax.experimental.pallas import tpu as pltpu
from jax.experimental.pallas import tpu_sc as plsc

_BATCH = 16384
_HIST = 50
_EMB = 32

_NC = 2
_NS = 16
_NW = _NC * _NS
_BPW = _BATCH // _NW
_K = 128
_NK = _BPW // _K

_mesh = plsc.VectorSubcoreMesh(core_axis_name="c", subcore_axis_name="s")


@functools.partial(
    pl.kernel,
    mesh=_mesh,
    out_type=jax.ShapeDtypeStruct((_HIST, _EMB, _BATCH), jnp.float32),
    compiler_params=pltpu.CompilerParams(
        use_tc_tiling_on_sc=False, needs_layout_passes=False),
    scratch_types=[
        pltpu.VMEM((_NK, _HIST, _K), jnp.int32),
        pltpu.VMEM((2, _BPW, _EMB), jnp.float32),
        pltpu.VMEM((2, _EMB, _BPW), jnp.float32),
        pltpu.SemaphoreType.DMA,
        pltpu.SemaphoreType.DMA,
        pltpu.SemaphoreType.DMA,
    ],
)
def _sc_gather(xt_hbm, table_hbm, out_hbm, idx_v, rows_v, tbuf_v, gsem, isem, osem):
    wid = lax.axis_index("s") * _NC + lax.axis_index("c")
    b0 = wid * _BPW

    for k in range(_NK):
        pltpu.async_copy(
            xt_hbm.at[:, pl.ds(b0 + k * _K, _K)], idx_v.at[k], isem)
    for k in range(_NK):
        pltpu.make_async_copy(
            xt_hbm.at[:, pl.ds(b0 + k * _K, _K)], idx_v.at[k], isem).wait()

    iota = lax.iota(jnp.int32, 16)

    def fire_g(h, s):
        for k in range(_NK):
            pltpu.async_copy(
                table_hbm.at[idx_v.at[k, h]],
                rows_v.at[s, pl.ds(k * _K, _K)], gsem)

    def wait_g(h, s):
        for k in range(_NK):
            pltpu.make_async_copy(
                table_hbm.at[idx_v.at[k, h]],
                rows_v.at[s, pl.ds(k * _K, _K)], gsem).wait()

    def fire_wb(h, s):
        pltpu.async_copy(tbuf_v.at[s], out_hbm.at[h, :, pl.ds(b0, _BPW)], osem)

    def wait_wb(h, s):
        pltpu.make_async_copy(
            tbuf_v.at[s], out_hbm.at[h, :, pl.ds(b0, _BPW)], osem).wait()

    def transpose(s):
        def vbody(v, carry):
            row = v * 16 + iota
            for c in range(_EMB):
                vec = plsc.load_gather(
                    rows_v.at[s], [row, jnp.full((16,), c, jnp.int32)])
                tbuf_v[s, c, pl.ds(v * 16, 16)] = vec
            return carry
        lax.fori_loop(0, _BPW // 16, vbody, 0)

    fire_g(0, 0)
    fire_g(1, 1)
    for h in (0, 1):
        wait_g(h, h)
        transpose(h)
        fire_g(h + 2, h)
        fire_wb(h, h)

    def body(t, carry):
        for s in range(2):
            h = 2 * t + s
            wait_g(h, s)
            wait_wb(h - 2, s)
            transpose(s)
            fire_g(h + 2, s)
            fire_wb(h, s)
        return carry

    lax.fori_loop(1, _HIST // 2 - 1, body, 0)

    for s in range(2):
        h = _HIST - 2 + s
        wait_g(h, s)
        wait_wb(h - 2, s)
        transpose(s)
        fire_wb(h, s)
    wait_wb(_HIST - 2, 0)
    wait_wb(_HIST - 1, 1)


def _mask_body(xt_ref, o_ref):
    o_ref[...] = xt_ref[...] > 0


_mask_call = pl.pallas_call(
    _mask_body,
    out_shape=jax.ShapeDtypeStruct((_HIST, _BATCH), jnp.bool_),
    grid=(8,),
    in_specs=[pl.BlockSpec((_HIST, _BATCH // 8), lambda i: (0, i))],
    out_specs=pl.BlockSpec((_HIST, _BATCH // 8), lambda i: (0, i)),
)


def kernel(x, table):
    xt = x.T.astype(jnp.int32)
    out = _sc_gather(xt, table)
    embs = out.transpose(2, 0, 1)
    mask = _mask_call(xt).T
    return embs, mask


# speedup vs baseline: 2.2419x; 1.4965x over previous
import functools

import jax
import jax.numpy as jnp
from jax import lax
from jax.experimental import pallas as pl
from jax.experimental.pallas import tpu as pltpu
from jax.experimental.pallas import tpu_sc as plsc

_BATCH = 16384
_HIST = 50
_EMB = 32

_NC = 2
_NS = 16
_NW = _NC * _NS
_BPW = _BATCH // _NW
_K = 128
_NK = _BPW // _K

_mesh = plsc.VectorSubcoreMesh(core_axis_name="c", subcore_axis_name="s")


@functools.partial(
    pl.kernel,
    mesh=_mesh,
    out_type=jax.ShapeDtypeStruct((_HIST, _EMB, _BATCH), jnp.float32),
    compiler_params=pltpu.CompilerParams(
        use_tc_tiling_on_sc=False, needs_layout_passes=False),
    scratch_types=[
        pltpu.VMEM((_NK, _HIST, _K), jnp.int32),
        pltpu.VMEM((2, _BPW, _EMB), jnp.float32),
        pltpu.VMEM((2, _EMB, _BPW), jnp.float32),
        pltpu.SemaphoreType.DMA,
        pltpu.SemaphoreType.DMA,
        pltpu.SemaphoreType.DMA,
    ],
)
def _sc_gather(xt_hbm, table_hbm, out_hbm, idx_v, rows_v, tbuf_v, gsem, isem, osem):
    wid = lax.axis_index("s") * _NC + lax.axis_index("c")
    b0 = wid * _BPW

    for k in range(_NK):
        pltpu.async_copy(
            xt_hbm.at[:, pl.ds(b0 + k * _K, _K)], idx_v.at[k], isem)
    for k in range(_NK):
        pltpu.make_async_copy(
            xt_hbm.at[:, pl.ds(b0 + k * _K, _K)], idx_v.at[k], isem).wait()

    iota = lax.iota(jnp.int32, 16)

    def fire_g(h, s):
        for k in range(_NK):
            pltpu.async_copy(
                table_hbm.at[idx_v.at[k, h]],
                rows_v.at[s, pl.ds(k * _K, _K)], gsem)

    def wait_g(h, s):
        for k in range(_NK):
            pltpu.make_async_copy(
                table_hbm.at[idx_v.at[k, h]],
                rows_v.at[s, pl.ds(k * _K, _K)], gsem).wait()

    def fire_wb(h, s):
        pltpu.async_copy(tbuf_v.at[s], out_hbm.at[h, :, pl.ds(b0, _BPW)], osem)

    def wait_wb(h, s):
        pltpu.make_async_copy(
            tbuf_v.at[s], out_hbm.at[h, :, pl.ds(b0, _BPW)], osem).wait()

    def transpose(s):
        def vbody(v, carry):
            row = v * 16 + iota
            for c in range(_EMB):
                col = (c + iota) & (_EMB - 1)
                vec = plsc.load_gather(rows_v.at[s], [row, col])
                plsc.store_scatter(tbuf_v.at[s], [col, row], vec)
            return carry
        lax.fori_loop(0, _BPW // 16, vbody, 0)

    fire_g(0, 0)
    fire_g(1, 1)
    for h in (0, 1):
        wait_g(h, h)
        transpose(h)
        fire_g(h + 2, h)
        fire_wb(h, h)

    def body(t, carry):
        for s in range(2):
            h = 2 * t + s
            wait_g(h, s)
            wait_wb(h - 2, s)
            transpose(s)
            fire_g(h + 2, s)
            fire_wb(h, s)
        return carry

    lax.fori_loop(1, _HIST // 2 - 1, body, 0)

    for s in range(2):
        h = _HIST - 2 + s
        wait_g(h, s)
        wait_wb(h - 2, s)
        transpose(s)
        fire_wb(h, s)
    wait_wb(_HIST - 2, 0)
    wait_wb(_HIST - 1, 1)


def _mask_body(xt_ref, o_ref):
    o_ref[...] = xt_ref[...] > 0


_mask_call = pl.pallas_call(
    _mask_body,
    out_shape=jax.ShapeDtypeStruct((_HIST, _BATCH), jnp.bool_),
    grid=(8,),
    in_specs=[pl.BlockSpec((_HIST, _BATCH // 8), lambda i: (0, i))],
    out_specs=pl.BlockSpec((_HIST, _BATCH // 8), lambda i: (0, i)),
)


def kernel(x, table):
    xt = x.T.astype(jnp.int32)
    out = _sc_gather(xt, table)
    embs = out.transpose(2, 0, 1)
    mask = _mask_call(xt).T
    return embs, mask


# speedup vs baseline: 2.6234x; 1.1702x over previous
import functools

import jax
import jax.numpy as jnp
from jax import lax
from jax.experimental import pallas as pl
from jax.experimental.pallas import tpu as pltpu
from jax.experimental.pallas import tpu_sc as plsc

_BATCH = 16384
_HIST = 50
_EMB = 32

_NC = 2
_NS = 16
_NW = _NC * _NS
_BPW = _BATCH // _NW
_K = 128
_NK = _BPW // _K

_mesh = plsc.VectorSubcoreMesh(core_axis_name="c", subcore_axis_name="s")


@functools.partial(
    pl.kernel,
    mesh=_mesh,
    out_type=jax.ShapeDtypeStruct((_HIST, _EMB, _BATCH), jnp.float32),
    compiler_params=pltpu.CompilerParams(
        use_tc_tiling_on_sc=False, needs_layout_passes=False),
    scratch_types=[
        pltpu.VMEM((_NK, _HIST, _K), jnp.int32),
        pltpu.VMEM((2, _BPW, _EMB), jnp.float32),
        pltpu.VMEM((2, _EMB, _BPW), jnp.float32),
        pltpu.SemaphoreType.DMA,
        pltpu.SemaphoreType.DMA,
        pltpu.SemaphoreType.DMA,
    ],
)
def _sc_gather(xt_hbm, table_hbm, out_hbm, idx_v, rows_v, tbuf_v, gsem, isem, osem):
    wid = lax.axis_index("s") * _NC + lax.axis_index("c")
    b0 = wid * _BPW

    for k in range(_NK):
        pltpu.async_copy(
            xt_hbm.at[:, pl.ds(b0 + k * _K, _K)], idx_v.at[k], isem)
    for k in range(_NK):
        pltpu.make_async_copy(
            xt_hbm.at[:, pl.ds(b0 + k * _K, _K)], idx_v.at[k], isem).wait()

    iota = lax.iota(jnp.int32, 16)

    def fire_g(h, s):
        for k in range(_NK):
            pltpu.async_copy(
                table_hbm.at[idx_v.at[k, h]],
                rows_v.at[s, pl.ds(k * _K, _K)], gsem)

    def wait_g(h, s):
        for k in range(_NK):
            pltpu.make_async_copy(
                table_hbm.at[idx_v.at[k, h]],
                rows_v.at[s, pl.ds(k * _K, _K)], gsem).wait()

    def fire_wb(h, s):
        pltpu.async_copy(tbuf_v.at[s], out_hbm.at[h, :, pl.ds(b0, _BPW)], osem)

    def wait_wb(h, s):
        pltpu.make_async_copy(
            tbuf_v.at[s], out_hbm.at[h, :, pl.ds(b0, _BPW)], osem).wait()

    def transpose(s):
        @plsc.parallel_loop(0, _BPW // 16, unroll=2)
        def vbody(v):
            row = v * 16 + iota
            for c in range(_EMB):
                col = (c + iota) & (_EMB - 1)
                vec = plsc.load_gather(rows_v.at[s], [row, col])
                plsc.store_scatter(tbuf_v.at[s], [col, row], vec)

    fire_g(0, 0)
    fire_g(1, 1)
    for h in (0, 1):
        wait_g(h, h)
        transpose(h)
        fire_g(h + 2, h)
        fire_wb(h, h)

    def body(t, carry):
        for s in range(2):
            h = 2 * t + s
            wait_g(h, s)
            wait_wb(h - 2, s)
            transpose(s)
            fire_g(h + 2, s)
            fire_wb(h, s)
        return carry

    lax.fori_loop(1, _HIST // 2 - 1, body, 0)

    for s in range(2):
        h = _HIST - 2 + s
        wait_g(h, s)
        wait_wb(h - 2, s)
        transpose(s)
        fire_wb(h, s)
    wait_wb(_HIST - 2, 0)
    wait_wb(_HIST - 1, 1)


def _mask_body(xt_ref, o_ref):
    o_ref[...] = xt_ref[...] > 0


_mask_call = pl.pallas_call(
    _mask_body,
    out_shape=jax.ShapeDtypeStruct((_HIST, _BATCH), jnp.bool_),
    grid=(8,),
    in_specs=[pl.BlockSpec((_HIST, _BATCH // 8), lambda i: (0, i))],
    out_specs=pl.BlockSpec((_HIST, _BATCH // 8), lambda i: (0, i)),
)


def kernel(x, table):
    xt = x.T.astype(jnp.int32)
    out = _sc_gather(xt, table)
    embs = out.transpose(2, 0, 1)
    mask = _mask_call(xt).T
    return embs, mask
